# baseline (speedup 1.0000x reference)
import functools

import jax
import jax.numpy as jnp
from jax import lax
from jax.experimental import pallas as pl
from jax.experimental.pallas import tpu as pltpu
from jax.experimental.pallas import tpu_sc as plsc


def sc_add(x2, lid, table):
    rows, d = x2.shape
    nb = lid.shape[0]
    nw = 32
    rows_per = rows // nw
    ch = 16
    nch = rows_per // ch
    dblk = d // 16
    tiles_per_batch = nw // nb

    mesh = plsc.VectorSubcoreMesh(core_axis_name="c", subcore_axis_name="s")

    @functools.partial(
        pl.kernel,
        out_type=jax.ShapeDtypeStruct((rows, d), jnp.float32),
        mesh=mesh,
        scratch_types=[
            pltpu.VMEM((nb,), jnp.int32),
            pltpu.VMEM((nb, d), jnp.float32),
            pltpu.VMEM((2, ch, d), jnp.float32),
            pltpu.VMEM((2, ch, d), jnp.float32),
            pltpu.SemaphoreType.DMA,
            pltpu.SemaphoreType.DMA((2,)),
            pltpu.SemaphoreType.DMA((2,)),
        ],
    )
    def k(x_hbm, lid_hbm, tab_hbm, out_hbm, idx_v, emb_v, xin, xout, sem_e, sem_i, sem_o):
        cid = lax.axis_index("c")
        sid = lax.axis_index("s")
        wid = sid * 2 + cid
        base = wid * rows_per
        b = wid // tiles_per_batch

        pltpu.sync_copy(lid_hbm, idx_v)
        pltpu.async_copy(tab_hbm.at[idx_v], emb_v, sem_e).wait()

        def start_in(c, j):
            return pltpu.async_copy(
                x_hbm.at[pl.ds(base + c * ch, ch)], xin.at[j], sem_i.at[j])

        def start_out(c, j):
            return pltpu.async_copy(
                xout.at[j], out_hbm.at[pl.ds(base + c * ch, ch)], sem_o.at[j])

        def compute(j):
            def db_body(db, _):
                ev = emb_v[b, pl.ds(db * 16, 16)]

                def r_body(r):
                    xout[j, r, pl.ds(db * 16, 16)] = (
                        xin[j, r, pl.ds(db * 16, 16)] + ev)
                    return None

                plsc.parallel_loop(0, ch, 1, unroll=4)(r_body)
                return None

            lax.fori_loop(0, dblk, db_body, None)

        hin = [None] * nch
        hout = [None] * nch
        hin[0] = start_in(0, 0)
        hin[1] = start_in(1, 1)
        for c in range(nch):
            j = c % 2
            hin[c].wait()
            if c >= 2:
                hout[c - 2].wait()
            compute(j)
            hout[c] = start_out(c, j)
            if c + 2 < nch:
                hin[c + 2] = start_in(c + 2, j)
        hout[nch - 2].wait()
        hout[nch - 1].wait()

    return k(x2, lid, table)


def kernel(x, language_id, language_embeddings):
    batch, seq, d = x.shape
    x2 = x.reshape(batch * seq, d)
    out2 = sc_add(x2, language_id.astype(jnp.int32), language_embeddings)
    return out2.reshape(batch, seq, d)


# speedup vs baseline: 1.8344x; 1.8344x over previous
import jax
import jax.numpy as jnp
from jax.experimental import pallas as pl
from jax.experimental.pallas import tpu as pltpu


def kernel(x, language_id, language_embeddings):
    batch, seq, d = x.shape
    tile = 2048
    per_batch = seq // tile
    x2 = x.reshape(batch * seq, d)
    tab3 = language_embeddings[:, None, :]
    lid = language_id.astype(jnp.int32)

    def body(lid_ref, x_ref, e_ref, o_ref):
        o_ref[...] = x_ref[...] + e_ref[0]

    grid_spec = pltpu.PrefetchScalarGridSpec(
        num_scalar_prefetch=1,
        grid=(batch * per_batch,),
        in_specs=[
            pl.BlockSpec((tile, d), lambda i, lid_ref: (i, 0)),
            pl.BlockSpec((1, 1, d), lambda i, lid_ref: (lid_ref[i // per_batch], 0, 0)),
        ],
        out_specs=pl.BlockSpec((tile, d), lambda i, lid_ref: (i, 0)),
    )
    out2 = pl.pallas_call(
        body,
        grid_spec=grid_spec,
        out_shape=jax.ShapeDtypeStruct(x2.shape, x.dtype),
        compiler_params=pltpu.CompilerParams(
            dimension_semantics=("arbitrary",),
        ),
    )(lid, x2, tab3)
    return out2.reshape(batch, seq, d)


# speedup vs baseline: 1.8525x; 1.0099x over previous
import jax
import jax.numpy as jnp
from jax.experimental import pallas as pl
from jax.experimental.pallas import tpu as pltpu


def kernel(x, language_id, language_embeddings):
    batch, seq, d = x.shape
    v = language_embeddings.shape[0]
    tile = 2048
    per_batch = seq // tile
    x2 = x.reshape(batch * seq, d)
    tab3 = language_embeddings[:, None, :]
    lid = language_id.astype(jnp.int32)

    def body(x_ref, lid_ref, tab_ref, o_ref):
        i = pl.program_id(0)
        row = lid_ref[i // per_batch]
        o_ref[...] = x_ref[...] + tab_ref[row]

    out2 = pl.pallas_call(
        body,
        grid=(batch * per_batch,),
        in_specs=[
            pl.BlockSpec((tile, d), lambda i: (i, 0)),
            pl.BlockSpec(memory_space=pltpu.SMEM),
            pl.BlockSpec((v, 1, d), lambda i: (0, 0, 0)),
        ],
        out_specs=pl.BlockSpec((tile, d), lambda i: (i, 0)),
        out_shape=jax.ShapeDtypeStruct(x2.shape, x.dtype),
        compiler_params=pltpu.CompilerParams(
            dimension_semantics=("arbitrary",),
        ),
    )(x2, lid, tab3)
    return out2.reshape(batch, seq, d)
